# Initial kernel scaffold; baseline (speedup 1.0000x reference)
import jax
import jax.numpy as jnp
from jax.experimental import pallas as pl


def kernel(x, adj, bn_feat_g, bn_feat_b, W_feat, b_feat, bnc0_g, bnc0_b, Wc0, bc0, bnc1_g, bnc1_b, Wc1, bc1, bnc2_g, bnc2_b, Wc2, bc2, bnfc0_g, bnfc0_b, W_l0, b_l0, bn_h_g, bn_h_b, W_cls, b_cls):
    raise NotImplementedError("write your pallas kernel here")


# speedup vs baseline: 1902.4402x; 1902.4402x over previous
import jax
import jax.numpy as jnp
from jax.experimental import pallas as pl
from jax.experimental.pallas import tpu as pltpu


def _bn(h, g, b):
    m = jnp.mean(h, axis=0, keepdims=True)
    v = jnp.mean((h - m) ** 2, axis=0, keepdims=True)
    return (h - m) * jax.lax.rsqrt(v + 1e-5) * g + b


def _fwd_kernel(x_ref, adj_ref, bn_feat_g, bn_feat_b, W_feat, b_feat,
                bnc0_g, bnc0_b, Wc0, bc0,
                bnc1_g, bnc1_b, Wc1, bc1,
                bnc2_g, bnc2_b, Wc2, bc2,
                bnfc0_g, bnfc0_b, W_l0, b_l0,
                bn_h_g, bn_h_b, W_cls, b_cls,
                out_ref, h_ref):
    B, N, _ = adj_ref.shape

    dinv = jnp.concatenate(
        [jax.lax.rsqrt(jnp.sum(adj_ref[b], axis=1, keepdims=True) + 1.0)
         for b in range(B)], axis=0)

    h = _bn(x_ref[:], bn_feat_g[:], bn_feat_b[:])
    h_ref[:, :] = jnp.maximum(
        jnp.dot(h, W_feat[:], preferred_element_type=jnp.float32) + b_feat[:],
        0.0)

    for (g, bb, W, bias) in ((bnc0_g, bnc0_b, Wc0, bc0),
                             (bnc1_g, bnc1_b, Wc1, bc1),
                             (bnc2_g, bnc2_b, Wc2, bc2)):
        h = _bn(h_ref[:, :], g[:], bb[:])
        hw = jnp.dot(h, W[:], preferred_element_type=jnp.float32)
        v = dinv * hw
        outs = []
        for b in range(B):
            vb = v[b * N:(b + 1) * N]
            rb = jax.lax.dot_general(adj_ref[b], vb,
                                     (((0,), (0,)), ((), ())),
                                     preferred_element_type=jnp.float32)
            outs.append(rb + vb)
        r = jnp.concatenate(outs, axis=0)
        h_ref[:, :] = jnp.maximum(dinv * r + bias[:], 0.0)

    pooled = jnp.concatenate(
        [jnp.mean(h_ref[b * N:(b + 1) * N, :], axis=0, keepdims=True)
         for b in range(B)], axis=0)

    z = _bn(pooled, bnfc0_g[:], bnfc0_b[:])
    z = jnp.maximum(
        jnp.dot(z, W_l0[:], preferred_element_type=jnp.float32) + b_l0[:], 0.0)
    z = _bn(z, bn_h_g[:], bn_h_b[:])
    logits = jnp.dot(z, W_cls[:], preferred_element_type=jnp.float32) + b_cls[:]
    lmax = jnp.max(logits, axis=1, keepdims=True)
    e = logits - lmax
    out_ref[:, :] = e - jnp.log(jnp.sum(jnp.exp(e), axis=1, keepdims=True))


def kernel(x, adj, bn_feat_g, bn_feat_b, W_feat, b_feat,
           bnc0_g, bnc0_b, Wc0, bc0,
           bnc1_g, bnc1_b, Wc1, bc1,
           bnc2_g, bnc2_b, Wc2, bc2,
           bnfc0_g, bnfc0_b, W_l0, b_l0,
           bn_h_g, bn_h_b, W_cls, b_cls):
    B, N, F = x.shape
    H = W_feat.shape[1]
    C = W_cls.shape[1]
    row = lambda a: a.reshape(1, -1)
    return pl.pallas_call(
        _fwd_kernel,
        out_shape=jax.ShapeDtypeStruct((B, C), jnp.float32),
        scratch_shapes=[pltpu.VMEM((B * N, H), jnp.float32)],
    )(x.reshape(B * N, F), adj,
      row(bn_feat_g), row(bn_feat_b), W_feat, row(b_feat),
      row(bnc0_g), row(bnc0_b), Wc0, row(bc0),
      row(bnc1_g), row(bnc1_b), Wc1, row(bc1),
      row(bnc2_g), row(bnc2_b), Wc2, row(bc2),
      row(bnfc0_g), row(bnfc0_b), W_l0, row(b_l0),
      row(bn_h_g), row(bn_h_b), W_cls, row(b_cls))


# speedup vs baseline: 2084.7875x; 1.0958x over previous
import jax
import jax.numpy as jnp
from jax.experimental import pallas as pl
from jax.experimental.pallas import tpu as pltpu


def _stats(h):
    m = jnp.mean(h, axis=0, keepdims=True)
    sq = jnp.mean(h * h, axis=0, keepdims=True)
    return m, jax.lax.rsqrt(jnp.maximum(sq - m * m, 0.0) + 1e-5)


def _bn(h, g, b):
    m, isd = _stats(h)
    return (h - m) * isd * g + b


def _fwd_kernel(x_ref, adj_ref, bn_feat_g, bn_feat_b, W_feat, b_feat,
                bnc0_g, bnc0_b, Wc0, bc0,
                bnc1_g, bnc1_b, Wc1, bc1,
                bnc2_g, bnc2_b, Wc2, bc2,
                bnfc0_g, bnfc0_b, W_l0, b_l0,
                bn_h_g, bn_h_b, W_cls, b_cls,
                out_ref, h_ref, s_ref):
    B, N, _ = adj_ref.shape

    ii = jax.lax.broadcasted_iota(jnp.int32, (N, N), 0)
    jj = jax.lax.broadcasted_iota(jnp.int32, (N, N), 1)
    eye = (ii == jj).astype(jnp.float32)
    for b in range(B):
        at = adj_ref[b] + eye
        dinv = jax.lax.rsqrt(jnp.sum(at, axis=1, keepdims=True))
        s_ref[b] = at * dinv * dinv.reshape(1, N)

    x = x_ref[:]
    m, isd = _stats(x)
    srow = isd * bn_feat_g[:]
    trow = bn_feat_b[:] - m * srow
    Wp = W_feat[:] * srow.reshape(-1, 1)
    brow = (jnp.dot(trow, W_feat[:], preferred_element_type=jnp.float32)
            + b_feat[:])
    h_ref[:, :] = jnp.maximum(
        jnp.dot(x, Wp, preferred_element_type=jnp.float32) + brow, 0.0)

    for (g, bb, W, bias) in ((bnc0_g, bnc0_b, Wc0, bc0),
                             (bnc1_g, bnc1_b, Wc1, bc1),
                             (bnc2_g, bnc2_b, Wc2, bc2)):
        h = h_ref[:, :]
        m, isd = _stats(h)
        srow = isd * g[:]
        trow = bb[:] - m * srow
        Wp = W[:] * srow.reshape(-1, 1)
        brow = jnp.dot(trow, W[:], preferred_element_type=jnp.float32)
        hw = jnp.dot(h, Wp, preferred_element_type=jnp.float32) + brow
        for b in range(B):
            ob = jax.lax.dot_general(s_ref[b], hw[b * N:(b + 1) * N],
                                     (((0,), (0,)), ((), ())),
                                     preferred_element_type=jnp.float32)
            h_ref[b * N:(b + 1) * N, :] = jnp.maximum(ob + bias[:], 0.0)

    pooled = jnp.concatenate(
        [jnp.mean(h_ref[b * N:(b + 1) * N, :], axis=0, keepdims=True)
         for b in range(B)], axis=0)

    z = _bn(pooled, bnfc0_g[:], bnfc0_b[:])
    z = jnp.maximum(
        jnp.dot(z, W_l0[:], preferred_element_type=jnp.float32) + b_l0[:], 0.0)
    z = _bn(z, bn_h_g[:], bn_h_b[:])
    logits = jnp.dot(z, W_cls[:], preferred_element_type=jnp.float32) + b_cls[:]
    e = logits - jnp.max(logits, axis=1, keepdims=True)
    out_ref[:, :] = e - jnp.log(jnp.sum(jnp.exp(e), axis=1, keepdims=True))


def kernel(x, adj, bn_feat_g, bn_feat_b, W_feat, b_feat,
           bnc0_g, bnc0_b, Wc0, bc0,
           bnc1_g, bnc1_b, Wc1, bc1,
           bnc2_g, bnc2_b, Wc2, bc2,
           bnfc0_g, bnfc0_b, W_l0, b_l0,
           bn_h_g, bn_h_b, W_cls, b_cls):
    B, N, F = x.shape
    H = W_feat.shape[1]
    C = W_cls.shape[1]
    row = lambda a: a.reshape(1, -1)
    return pl.pallas_call(
        _fwd_kernel,
        out_shape=jax.ShapeDtypeStruct((B, C), jnp.float32),
        scratch_shapes=[pltpu.VMEM((B * N, H), jnp.float32),
                        pltpu.VMEM((B, N, N), jnp.float32)],
    )(x.reshape(B * N, F), adj,
      row(bn_feat_g), row(bn_feat_b), W_feat, row(b_feat),
      row(bnc0_g), row(bnc0_b), Wc0, row(bc0),
      row(bnc1_g), row(bnc1_b), Wc1, row(bc1),
      row(bnc2_g), row(bnc2_b), Wc2, row(bc2),
      row(bnfc0_g), row(bnfc0_b), W_l0, row(b_l0),
      row(bn_h_g), row(bn_h_b), W_cls, row(b_cls))
